# Initial kernel scaffold; baseline (speedup 1.0000x reference)
import jax
import jax.numpy as jnp
from jax.experimental import pallas as pl


def kernel(input, W1, b1, W2, b2, W3, b3):
    raise NotImplementedError("write your pallas kernel here")


# speedup vs baseline: 1.2208x; 1.2208x over previous
import functools

import jax
import jax.numpy as jnp
from jax.experimental import pallas as pl
from jax.experimental.pallas import tpu as pltpu

_BLOCK = 4096


def _mlp_kernel(x_ref, w1_ref, b1_ref, w2_ref, b2_ref, w3_ref, b3_ref, o_ref):
    dims = (((1,), (1,)), ((), ()))
    x = x_ref[...]
    h = jax.lax.dot_general(x, w1_ref[...], dims,
                            preferred_element_type=jnp.float32)
    h = jnp.maximum(h + b1_ref[...], 0.0)
    h = jax.lax.dot_general(h, w2_ref[...], dims,
                            preferred_element_type=jnp.float32)
    h = jnp.maximum(h + b2_ref[...], 0.0)
    h = jax.lax.dot_general(h, w3_ref[...], dims,
                            preferred_element_type=jnp.float32)
    o_ref[...] = h + b3_ref[...]


@functools.partial(jax.jit, static_argnames=())
def kernel(input, W1, b1, W2, b2, W3, b3):
    n, d = input.shape
    grid = (pl.cdiv(n, _BLOCK),)
    row_spec = pl.BlockSpec((_BLOCK, d), lambda i: (i, 0))
    full_spec = pl.BlockSpec((d, d), lambda i: (0, 0))
    bias_spec = pl.BlockSpec((1, d), lambda i: (0, 0))
    return pl.pallas_call(
        _mlp_kernel,
        grid=grid,
        in_specs=[row_spec, full_spec, bias_spec, full_spec, bias_spec,
                  full_spec, bias_spec],
        out_specs=row_spec,
        out_shape=jax.ShapeDtypeStruct((n, d), input.dtype),
        compiler_params=pltpu.CompilerParams(
            dimension_semantics=("arbitrary",),
        ),
    )(input, W1, b1.reshape(1, d), W2, b2.reshape(1, d), W3,
      b3.reshape(1, d))
